# baseline (speedup 1.0000x reference)
import functools
import jax
import jax.numpy as jnp
from jax import lax
from jax.experimental import pallas as pl
from jax.experimental.pallas import tpu as pltpu

N = 10000
E = 320000
D = 128
SG = 16
HL = 128

_NPAD = 10240


def _leaky(x):
    return jnp.where(x > 0, x, 0.01 * x)


def _mm_body(x_ref, w_ref, o_ref, *, act):
    y = jnp.dot(x_ref[...], w_ref[...], preferred_element_type=jnp.float32)
    if act:
        y = _leaky(y)
    o_ref[...] = y


def _mm_res_body(x_ref, w_ref, r_ref, o_ref, *, act):
    y = r_ref[...] + jnp.dot(x_ref[...], w_ref[...], preferred_element_type=jnp.float32)
    if act:
        y = _leaky(y)
    o_ref[...] = y


def _mm(x, w, res=None, act=True, br=1024):
    rows = x.shape[0]
    assert rows % br == 0, (rows, br)
    k = x.shape[1]
    n = w.shape[1]
    grid = rows // br
    in_specs = [
        pl.BlockSpec((br, k), lambda i: (i, 0)),
        pl.BlockSpec((k, n), lambda i: (0, 0)),
    ]
    args = [x, w]
    if res is not None:
        in_specs.append(pl.BlockSpec((br, n), lambda i: (i, 0)))
        args.append(res)
        body = functools.partial(_mm_res_body, act=act)
    else:
        body = functools.partial(_mm_body, act=act)
    return pl.pallas_call(
        body,
        grid=(grid,),
        in_specs=in_specs,
        out_specs=pl.BlockSpec((br, n), lambda i: (i, 0)),
        out_shape=jax.ShapeDtypeStruct((rows, n), jnp.float32),
    )(*args)


def _upd_body(h_ref, a_ref, b_ref, w_ref, o_ref):
    agg = a_ref[...] + b_ref[...]
    o_ref[...] = _leaky(h_ref[...] + jnp.dot(agg, w_ref[...], preferred_element_type=jnp.float32))


def _update(h, aggA, aggB, w, br=1024):
    rows = h.shape[0]
    grid = rows // br
    return pl.pallas_call(
        _upd_body,
        grid=(grid,),
        in_specs=[
            pl.BlockSpec((br, D), lambda i: (i, 0)),
            pl.BlockSpec((br, D), lambda i: (i, 0)),
            pl.BlockSpec((br, D), lambda i: (i, 0)),
            pl.BlockSpec((D, D), lambda i: (0, 0)),
        ],
        out_specs=pl.BlockSpec((br, D), lambda i: (i, 0)),
        out_shape=jax.ShapeDtypeStruct((rows, D), jnp.float32),
    )(h, aggA, aggB, w)


_EBR = 2560


def _eo_body(acc_ref, e8_ref, gt_ref, we_ref, ws_ref,
             eo_ref, loss_ref):
    i = pl.program_id(0)
    eo = _leaky(acc_ref[...] + jnp.dot(e8_ref[...], we_ref[...],
                                       preferred_element_type=jnp.float32))
    eo_ref[...] = eo
    logit = jnp.dot(eo, ws_ref[...], preferred_element_type=jnp.float32)[:, 0]
    s = jax.nn.sigmoid(logit)
    d = s - gt_ref[0, 0, :]
    part = jnp.sum(d * d)

    @pl.when(i == 0)
    def _():
        loss_ref[...] = jnp.zeros_like(loss_ref)

    loss_ref[...] += jnp.full((1, 1), 0.0, jnp.float32) + part


def _eo_head(eoacc, e8, gt3, we_e8, ws):
    grid = E // _EBR
    return pl.pallas_call(
        _eo_body,
        grid=(grid,),
        in_specs=[
            pl.BlockSpec((_EBR, D), lambda i: (i, 0)),
            pl.BlockSpec((_EBR, 8), lambda i: (i, 0)),
            pl.BlockSpec((1, 1, _EBR), lambda i: (i, 0, 0)),
            pl.BlockSpec((8, D), lambda i: (0, 0)),
            pl.BlockSpec((D, 1), lambda i: (0, 0)),
        ],
        out_specs=[
            pl.BlockSpec((_EBR, D), lambda i: (i, 0)),
            pl.BlockSpec((1, 1), lambda i: (0, 0)),
        ],
        out_shape=[
            jax.ShapeDtypeStruct((E, D), jnp.float32),
            jax.ShapeDtypeStruct((1, 1), jnp.float32),
        ],
    )(eoacc, e8, gt3, we_e8, ws)


def _vmean_body(sub_ref, agg_ref, wg_ref, vm_ref, s1_ref, s2_ref):
    i = pl.program_id(0)
    rows = _leaky(sub_ref[...] + jnp.dot(agg_ref[...], wg_ref[...],
                                         preferred_element_type=jnp.float32))
    vm = jnp.mean(rows.reshape(-1, SG, D), axis=1)
    vm_ref[...] = vm

    @pl.when(i == 0)
    def _():
        s1_ref[...] = jnp.zeros_like(s1_ref)
        s2_ref[...] = jnp.zeros_like(s2_ref)

    s1_ref[...] += jnp.sum(vm, axis=0, keepdims=True)
    s2_ref[...] += jnp.sum(vm * vm, axis=0, keepdims=True)


def _vmean_stats(sub, agg2, wg):
    grid = E // _EBR
    gb = _EBR // SG
    return pl.pallas_call(
        _vmean_body,
        grid=(grid,),
        in_specs=[
            pl.BlockSpec((_EBR, D), lambda i: (i, 0)),
            pl.BlockSpec((_EBR, D), lambda i: (i, 0)),
            pl.BlockSpec((D, D), lambda i: (0, 0)),
        ],
        out_specs=[
            pl.BlockSpec((gb, D), lambda i: (i, 0)),
            pl.BlockSpec((1, D), lambda i: (0, 0)),
            pl.BlockSpec((1, D), lambda i: (0, 0)),
        ],
        out_shape=[
            jax.ShapeDtypeStruct((E // SG, D), jnp.float32),
            jax.ShapeDtypeStruct((1, D), jnp.float32),
            jax.ShapeDtypeStruct((1, D), jnp.float32),
        ],
    )(sub, agg2, wg)


def _bnmm_body(x_ref, s1_ref, s2_ref, g_ref, b_ref, l_ref, bl_ref,
               y_ref, t1_ref, t2_ref, *, m, stats):
    i = pl.program_id(0)
    mu = s1_ref[...] / m
    var = s2_ref[...] / m - mu * mu
    rstd = jax.lax.rsqrt(var + 1e-5)
    xn = _leaky((x_ref[...] - mu) * rstd * g_ref[...] + b_ref[...])
    y = jnp.dot(xn, l_ref[...], preferred_element_type=jnp.float32) + bl_ref[...]
    y_ref[...] = y
    if stats:
        @pl.when(i == 0)
        def _():
            t1_ref[...] = jnp.zeros_like(t1_ref)
            t2_ref[...] = jnp.zeros_like(t2_ref)

        t1_ref[...] += jnp.sum(y, axis=0, keepdims=True)
        t2_ref[...] += jnp.sum(y * y, axis=0, keepdims=True)


def _bn_mm(x, s1, s2, g, b, L, bL, stats=True, br=2000):
    rows, k = x.shape
    n = L.shape[1]
    grid = rows // br
    outs = [jax.ShapeDtypeStruct((rows, n), jnp.float32)]
    out_specs = [pl.BlockSpec((br, n), lambda i: (i, 0))]
    if stats:
        outs += [jax.ShapeDtypeStruct((1, n), jnp.float32)] * 2
        out_specs += [pl.BlockSpec((1, n), lambda i: (0, 0))] * 2
    else:
        outs += [jax.ShapeDtypeStruct((1, 1), jnp.float32)] * 2
        out_specs += [pl.BlockSpec((1, 1), lambda i: (0, 0))] * 2
    res = pl.pallas_call(
        functools.partial(_bnmm_body, m=float(rows), stats=stats),
        grid=(grid,),
        in_specs=[
            pl.BlockSpec((br, k), lambda i: (i, 0)),
            pl.BlockSpec((1, k), lambda i: (0, 0)),
            pl.BlockSpec((1, k), lambda i: (0, 0)),
            pl.BlockSpec((1, k), lambda i: (0, 0)),
            pl.BlockSpec((1, k), lambda i: (0, 0)),
            pl.BlockSpec((k, n), lambda i: (0, 0)),
            pl.BlockSpec((1, n), lambda i: (0, 0)),
        ],
        out_specs=out_specs,
        out_shape=outs,
    )(x, s1, s2, g, b, L, bL)
    return res


def _seg_main(t, eproj, src, dst):
    m = _leaky(t[src] + eproj)
    agg = jax.ops.segment_sum(m, dst, num_segments=_NPAD)
    return agg, jnp.zeros_like(agg)


def _eoacc(A, Bm, src, dst):
    return A[src] + Bm[dst]


def _subgather(eo, sidx):
    return eo[sidx]


def _seg_global(sub, sep):
    agg = jax.ops.segment_sum(sub[sep[0]], sep[1], num_segments=E)
    agg = agg + jax.ops.segment_sum(sub[sep[1]], sep[0], num_segments=E)
    return agg


def kernel(node_features, actions, edge_index, angles, sub_graphs, sep_subgraphs,
           gt_edges, post_data, params):
    src, dst = edge_index[0], edge_index[1]
    e4 = jnp.concatenate([actions, angles], axis=-1)
    e8 = jnp.pad(e4, ((0, 0), (0, 4)))
    gt3 = gt_edges.reshape(E // _EBR, 1, _EBR)
    x = jnp.pad(node_features, ((0, _NPAD - N), (0, 0)))
    sidx = sub_graphs[0]
    sep = sep_subgraphs[0]

    def branch(q, gg):
        Wm_h, Wm_e = q['Wm'][:D], q['Wm'][D:]
        We_s, We_d, We_e = q['We'][:D], q['We'][D:2 * D], q['We'][2 * D:]
        wm_e8 = jnp.pad(Wm_e, ((0, 8 - Wm_e.shape[0]), (0, 0)))
        we_e8 = jnp.pad(We_e, ((0, 8 - We_e.shape[0]), (0, 0)))

        eproj = _mm(e8, wm_e8, act=False, br=_EBR)
        h = _mm(x, q['Wn0'], act=True)
        for _ in range(3):
            t = _mm(h, Wm_h, act=False)
            aggA, aggB = _seg_main(t, eproj, src, dst)
            h = _update(h, aggA, aggB, q['Wu'])
        A = _mm(h, We_s, act=False)
        Bm = _mm(h, We_d, act=False)
        eoacc = _eoacc(A, Bm, src, dst)
        eo, loss_sum = _eo_head(eoacc, e8, gt3, we_e8, q['ws'])
        return eo, loss_sum[0, 0] / E

    def head(eo, gg, v):
        sub = _subgather(eo, sidx)
        agg2 = _seg_global(sub, sep)
        vm, s1, s2 = _vmean_stats(sub, agg2, gg['Wg'])
        y1, t1, t2 = _bn_mm(vm, s1, s2, v['g1'].reshape(1, -1), v['b1'].reshape(1, -1),
                            v['L1'], v['bL1'].reshape(1, -1), stats=True)
        y2, u1, u2 = _bn_mm(y1, t1, t2, v['g2'].reshape(1, -1), v['b2'].reshape(1, -1),
                            v['L2'], v['bL2'].reshape(1, -1), stats=True)
        y3, _, _ = _bn_mm(y2, u1, u2, v['g3'].reshape(1, -1), v['b3'].reshape(1, -1),
                          v['L3'], v['bL3'].reshape(1, -1), stats=False)
        return y3[:, 0]

    eo1, l1 = branch(params['q1'], params['gg1'])
    eo2, l2 = branch(params['q2'], params['gg2'])
    v1 = head(eo1, params['gg1'], params['v1'])
    v2 = head(eo2, params['gg2'], params['v2'])
    return (v1, v2, (l1 + l2) / 4)
